# baseline (speedup 1.0000x reference)
import functools

import jax
import jax.numpy as jnp
from jax import lax
from jax.experimental import pallas as pl
from jax.experimental.pallas import tpu as pltpu
from jax.experimental.pallas import tpu_sc as plsc

B = 16384
D = 32

_NC, _NS = 2, 16
_NW = _NC * _NS
_BPW = B // _NW
_CHUNK = 128
_NCH = _BPW // _CHUNK


def _sc_gather(user_idx2d, item_idx2d, eug, eum, eig, eim):
  mesh = plsc.VectorSubcoreMesh(core_axis_name="c", subcore_axis_name="s")

  row_t = jax.ShapeDtypeStruct((B, D), jnp.float32)

  @functools.partial(
      pl.kernel,
      mesh=mesh,
      out_type=[row_t, row_t, row_t, row_t],
      compiler_params=pltpu.CompilerParams(use_tc_tiling_on_sc=False),
      scratch_types=[
          pltpu.VMEM((_NCH, _CHUNK), jnp.int32),
          pltpu.VMEM((_NCH, _CHUNK), jnp.int32),
          pltpu.VMEM((_BPW, D), jnp.float32),
          pltpu.VMEM((_BPW, D), jnp.float32),
          pltpu.VMEM((_BPW, D), jnp.float32),
          pltpu.VMEM((_BPW, D), jnp.float32),
          pltpu.SemaphoreType.DMA,
      ],
  )
  def k(uidx_hbm, iidx_hbm, eug_hbm, eum_hbm, eig_hbm, eim_hbm,
        ug_o, um_o, ig_o, im_o,
        uidx_v, iidx_v, ug_v, um_v, ig_v, im_v, sem):
    wid = lax.axis_index("s") * _NC + lax.axis_index("c")
    base = wid * _BPW
    crow = wid * _NCH
    pltpu.sync_copy(uidx_hbm.at[pl.ds(crow, _NCH)], uidx_v)
    pltpu.sync_copy(iidx_hbm.at[pl.ds(crow, _NCH)], iidx_v)
    copies = []
    for c in range(_NCH):
      sl = pl.ds(c * _CHUNK, _CHUNK)
      copies.append(pltpu.async_copy(eug_hbm.at[uidx_v.at[c]], ug_v.at[sl], sem))
      copies.append(pltpu.async_copy(eum_hbm.at[uidx_v.at[c]], um_v.at[sl], sem))
      copies.append(pltpu.async_copy(eig_hbm.at[iidx_v.at[c]], ig_v.at[sl], sem))
      copies.append(pltpu.async_copy(eim_hbm.at[iidx_v.at[c]], im_v.at[sl], sem))
    for cp in copies:
      cp.wait()
    out_sl = pl.ds(base, _BPW)
    pltpu.sync_copy(ug_v, ug_o.at[out_sl])
    pltpu.sync_copy(um_v, um_o.at[out_sl])
    pltpu.sync_copy(ig_v, ig_o.at[out_sl])
    pltpu.sync_copy(im_v, im_o.at[out_sl])

  return k(user_idx2d, item_idx2d, eug, eum, eig, eim)


def _tc_dense_body(ug_r, ig_r, um_r, im_r, gw_r, gb_r, w1a_r, w1b_r, b1_r,
                   w2_r, b2_r, w3_r, b3_r, w4_r, b4_r, out_r):
  ug = ug_r[...]
  ig = ig_r[...]
  gmf_logit = jnp.sum(ug * ig * gw_r[...], axis=1, keepdims=True) + gb_r[0, 0]
  h = jnp.maximum(
      jnp.dot(um_r[...], w1a_r[...], preferred_element_type=jnp.float32)
      + jnp.dot(im_r[...], w1b_r[...], preferred_element_type=jnp.float32)
      + b1_r[...], 0.0)
  h = jnp.maximum(
      jnp.dot(h, w2_r[...], preferred_element_type=jnp.float32) + b2_r[...], 0.0)
  h = jnp.maximum(
      jnp.dot(h, w3_r[...], preferred_element_type=jnp.float32) + b3_r[...], 0.0)
  mlp_logit = jnp.sum(h * w4_r[...], axis=1, keepdims=True) + b4_r[0, 0]
  out_r[...] = 0.5 * (jax.nn.sigmoid(gmf_logit) + jax.nn.sigmoid(mlp_logit))


def kernel(user_indices, item_indices, emb_user_gmf, emb_user_mlp,
           emb_item_gmf, emb_item_mlp, gmf_w, gmf_b, w1, b1, w2, b2, w3, b3,
           w4, b4):
  uidx = jnp.asarray(user_indices, jnp.int32).reshape(B // _CHUNK, _CHUNK)
  iidx = jnp.asarray(item_indices, jnp.int32).reshape(B // _CHUNK, _CHUNK)

  ug, um, ig, im = _sc_gather(uidx, iidx, emb_user_gmf, emb_user_mlp,
                              emb_item_gmf, emb_item_mlp)

  gw = gmf_w.reshape(1, D)
  gb = gmf_b.reshape(1, 1)
  w1a = w1[:D]
  w1b = w1[D:]
  b1r = b1.reshape(1, -1)
  b2r = b2.reshape(1, -1)
  b3r = b3.reshape(1, -1)
  w4r = w4.reshape(1, -1)
  b4r = b4.reshape(1, 1)

  blk = 4096
  grid = B // blk

  def row_spec():
    return pl.BlockSpec((blk, D), lambda i: (i, 0))

  def full_spec(shape):
    return pl.BlockSpec(shape, lambda i: tuple(0 for _ in shape))

  out = pl.pallas_call(
      _tc_dense_body,
      grid=(grid,),
      in_specs=[
          row_spec(), row_spec(), row_spec(), row_spec(),
          full_spec(gw.shape), full_spec(gb.shape),
          full_spec(w1a.shape), full_spec(w1b.shape), full_spec(b1r.shape),
          full_spec(w2.shape), full_spec(b2r.shape),
          full_spec(w3.shape), full_spec(b3r.shape),
          full_spec(w4r.shape), full_spec(b4r.shape),
      ],
      out_specs=pl.BlockSpec((blk, 1), lambda i: (i, 0)),
      out_shape=jax.ShapeDtypeStruct((B, 1), jnp.float32),
  )(ug, ig, um, im, gw, gb, w1a, w1b, b1r, w2, b2r, w3, b3r, w4r, b4r)
  return out
